# baseline (speedup 1.0000x reference)
import jax
import jax.numpy as jnp
from jax.experimental import pallas as pl
from jax.experimental.pallas import tpu as pltpu


def kernel(flat, cu_seqlens):
    B = cu_seqlens.shape[0] - 1
    D = flat.shape[1]

    def body(cu_ref, flat_ref, out_ref, idx_smem, sem, idx_sem):
        pltpu.make_async_copy(cu_ref, idx_smem, idx_sem).start()
        pltpu.make_async_copy(cu_ref, idx_smem, idx_sem).wait()
        copies = [
            pltpu.make_async_copy(
                flat_ref.at[pl.ds(idx_smem[i], 1), :],
                out_ref.at[pl.ds(i, 1), :],
                sem,
            )
            for i in range(B)
        ]
        for c in copies:
            c.start()
        for c in copies:
            c.wait()

    return pl.pallas_call(
        body,
        in_specs=[
            pl.BlockSpec(memory_space=pltpu.MemorySpace.HBM),
            pl.BlockSpec(memory_space=pltpu.MemorySpace.HBM),
        ],
        out_specs=pl.BlockSpec((B, D), memory_space=pltpu.MemorySpace.VMEM),
        scratch_shapes=[
            pltpu.SMEM((B + 1,), jnp.int32),
            pltpu.SemaphoreType.DMA,
            pltpu.SemaphoreType.DMA,
        ],
        out_shape=jax.ShapeDtypeStruct((B, D), jnp.float32),
    )(cu_seqlens, flat)
